# Initial kernel scaffold; baseline (speedup 1.0000x reference)
import jax
import jax.numpy as jnp
from jax.experimental import pallas as pl


def kernel(rel_id, tau, W_trend, A, mu, s):
    raise NotImplementedError("write your pallas kernel here")


# speedup vs baseline: 2.9924x; 2.9924x over previous
import functools

import jax
import jax.numpy as jnp
from jax import lax
from jax.experimental import pallas as pl
from jax.experimental.pallas import tpu as pltpu
from jax.experimental.pallas import tpu_sc as plsc

SIGMA_MIN = 0.02
SIGMA_MAX = 0.3
EPS = 1e-09

NC = 2
NS = 16
L = 16
NW = NC * NS


def _encoder_call(rel_id, tau, W_trend, A2, ms):
    B = rel_id.shape[0]
    DIM = W_trend.shape[1]
    K = ms.shape[1] // 2
    AF = A2.shape[1]
    BPW = B // NW
    C = 128
    NCHUNK = BPW // C
    HG = DIM // L

    mesh = plsc.VectorSubcoreMesh(
        core_axis_name="c", subcore_axis_name="s",
        num_cores=NC, num_subcores=NS)

    f32 = jnp.float32

    @functools.partial(
        pl.kernel,
        out_type=(
            jax.ShapeDtypeStruct((B, DIM), f32),
            jax.ShapeDtypeStruct((B, DIM), f32),
            jax.ShapeDtypeStruct((B, DIM), f32),
        ),
        mesh=mesh,
        compiler_params=pltpu.CompilerParams(use_tc_tiling_on_sc=False),
        scratch_types=[
            pltpu.VMEM((C,), jnp.int32), pltpu.VMEM((C,), jnp.int32),
            pltpu.VMEM((C,), f32), pltpu.VMEM((C,), f32),
            pltpu.VMEM((C, DIM), f32), pltpu.VMEM((C, DIM), f32),
            pltpu.VMEM((C, AF), f32), pltpu.VMEM((C, AF), f32),
            pltpu.VMEM((C, 2 * K), f32), pltpu.VMEM((C, 2 * K), f32),
            pltpu.VMEM((C, L), f32),
            pltpu.VMEM((C, DIM), f32),
            pltpu.VMEM((C, DIM), f32),
            pltpu.VMEM((C, DIM), f32),
            pltpu.SemaphoreType.DMA,
            pltpu.SemaphoreType.DMA,
        ],
    )
    def enc(rel_hbm, tau_hbm, w_hbm, a_hbm, ms_hbm,
            de_hbm, dt_hbm, dp_hbm,
            idx0, idx1, tv0, tv1, wv0, wv1, av0, av1,
            msv0, msv1, gv, oe, ot, op, sem0, sem1):
        wid = lax.axis_index("s") * NC + lax.axis_index("c")
        base = wid * BPW
        bufs = ((idx0, tv0, wv0, av0, msv0, sem0),
                (idx1, tv1, wv1, av1, msv1, sem1))

        def start(c, slot):
            idxv, tv, wv, av, msv, sem = bufs[slot]
            off = base + c * C
            pltpu.sync_copy(rel_hbm.at[pl.ds(off, C)], idxv)
            pltpu.sync_copy(tau_hbm.at[pl.ds(off, C)], tv)
            return (
                pltpu.async_copy(w_hbm.at[idxv], wv, sem),
                pltpu.async_copy(a_hbm.at[idxv], av, sem),
                pltpu.async_copy(ms_hbm.at[idxv], msv, sem),
            )

        def compute(slot):
            idxv, tv, wv, av, msv, _ = bufs[slot]

            def gbody(g, carry):
                b0 = g * L
                tvec = tv[pl.ds(b0, L)]
                for i in range(L):
                    b = b0 + i
                    row = msv[b, :]
                    sig = SIGMA_MIN + (SIGMA_MAX - SIGMA_MIN) / (
                        1.0 + jnp.exp(-row))
                    den = 2.0 * sig * sig + EPS
                    d = tvec[i] - row
                    q = lax.rev(d * d, (0,)) / den
                    gv[b, :] = lax.rev(jnp.exp(-q), (0,))
                return carry

            lax.fori_loop(0, C // L, gbody, 0)

            def obody(g, carry):
                b0 = g * L
                tvec = tv[pl.ds(b0, L)]
                for i in range(L):
                    b = b0 + i
                    tau_b = tvec[i]
                    gvec = gv[b, :]
                    for h in range(HG):
                        w16 = wv[b, pl.ds(h * L, L)]
                        dth = w16 * tau_b
                        acc0 = av[b, pl.ds(h * L, L)] * gvec[0]
                        acc1 = av[b, pl.ds(DIM + h * L, L)] * gvec[1]
                        for k in range(2, K):
                            a16 = av[b, pl.ds(k * DIM + h * L, L)]
                            if k % 2 == 0:
                                acc0 = acc0 + a16 * gvec[k]
                            else:
                                acc1 = acc1 + a16 * gvec[k]
                        acc = acc0 + acc1
                        ot[b, pl.ds(h * L, L)] = dth
                        op[b, pl.ds(h * L, L)] = acc
                        oe[b, pl.ds(h * L, L)] = dth + acc
                return carry

            lax.fori_loop(0, C // L, obody, 0)

        pend = start(0, 0)
        for c in range(NCHUNK):
            slot = c & 1
            cur = pend
            if c + 1 < NCHUNK:
                pend = start(c + 1, 1 - slot)
            for cp in cur:
                cp.wait()
            compute(slot)
            off = base + c * C
            pltpu.sync_copy(oe, de_hbm.at[pl.ds(off, C)])
            pltpu.sync_copy(ot, dt_hbm.at[pl.ds(off, C)])
            pltpu.sync_copy(op, dp_hbm.at[pl.ds(off, C)])

    return enc(rel_id, tau, W_trend, A2, ms)


def kernel(rel_id, tau, W_trend, A, mu, s):
    N, K, DIM = A.shape
    A2 = A.reshape(N, K * DIM)
    ms = jnp.concatenate([mu, s[:, ::-1]], axis=1)
    de, dt, dp = _encoder_call(rel_id.astype(jnp.int32), tau,
                               W_trend, A2, ms)
    return (de, dt, dp)
